# baseline (speedup 1.0000x reference)
import functools

import jax
import jax.numpy as jnp
from jax.experimental import pallas as pl


def _body(nk, nt, d_ref, v_ref, v1_ref, w1a_ref, w1bl_ref, w2_ref, b1_ref,
          b2_ref, o_ref):
    v = v_ref[...]
    v1 = v1_ref[...]
    flags = [jnp.zeros((), v.dtype) - v]
    for d in range(1, nk):
        src = v if d % 2 == 0 else v1
        e = d if d % 2 == 0 else d - 1
        if e == 0:
            shifted = src
        else:
            shifted = jnp.concatenate(
                [jnp.full((e, nt), 0x7FFF, v.dtype), src[: nk - e, :]],
                axis=0)
        flags.append(jnp.zeros((), v.dtype) - (v ^ shifted))
    while len(flags) > 1:
        flags = [a & b for a, b in zip(flags[::2], flags[1::2])] + (
            [flags[-1]] if len(flags) % 2 else [])
    is_new = jnp.where(flags[0].astype(jnp.int32) < 0, 1.0, 0.0)

    a = jnp.dot(w1a_ref[...], d_ref[...], preferred_element_type=jnp.float32)
    b = jnp.dot(w1bl_ref[...], is_new, preferred_element_type=jnp.float32)
    h = jnp.tanh(a + b + b1_ref[...])
    logits = jnp.dot(w2_ref[...], h,
                     preferred_element_type=jnp.float32) + b2_ref[...]
    m = jnp.max(logits, axis=0, keepdims=True)
    e = jnp.exp(logits - m)
    o_ref[...] = e / jnp.sum(e, axis=0, keepdims=True)


def kernel(distances, values, W1, b1, W2, b2):
    B, S, K = distances.shape
    T = B * S
    HID = W1.shape[1]
    OUT = W2.shape[1]
    OUTP = 8

    dT = distances.reshape(T, K).T
    vT = values.astype(jnp.int16).reshape(T, K).T
    fill = jnp.full((1, T), 0x7FFF, jnp.int16)
    vT1 = jnp.concatenate([fill, vT[:-1]], axis=0)

    w1aT = W1[:K].T
    L = jnp.tril(jnp.ones((K, K), jnp.float32))
    w1blT = W1[K:].T @ L
    w2T = jnp.zeros((OUTP, HID), jnp.float32).at[:OUT].set(W2.T)
    b1c = b1.reshape(HID, 1)
    b2c = jnp.full((OUTP, 1), -1e9, jnp.float32).at[:OUT, 0].set(b2)

    out = pl.pallas_call(
        functools.partial(_body, K, T),
        out_shape=jax.ShapeDtypeStruct((OUTP, T), jnp.float32),
    )(dT, vT, vT1, w1aT, w1blT, w2T, b1c, b2c)

    return out[:OUT].T.reshape(B, S, OUT)
